# baseline (speedup 1.0000x reference)
import functools

import jax
import jax.numpy as jnp
from jax import lax
from jax.experimental import pallas as pl
from jax.experimental.pallas import tpu as pltpu
from jax.experimental.pallas import tpu_sc as plsc

B = 4
S = 8192
D = 1024
K = 4096
C = 4
NC = K // C
H = 256
M = 128
CL = 10

NW = 32
PERM_PAD = B * K + 128


_RB = 512


def _norms_body(x_ref, o_ref):
    y = x_ref[...]
    yy = y * y
    c = yy.reshape(B, _RB, 8, 128)
    acc = c[:, :, 0, :]
    for i in range(1, 8):
        acc = acc + c[:, :, i, :]
    st = 64
    while st >= 1:
        acc = acc[..., :st] + acc[..., st:2 * st]
        st //= 2
    o_ref[...] = jnp.sqrt(acc[..., 0])


_norms_call = pl.pallas_call(
    _norms_body,
    grid=(S // _RB,),
    in_specs=[pl.BlockSpec((B, _RB, D), lambda i: (0, i, 0))],
    out_specs=pl.BlockSpec((B, _RB), lambda i: (0, i)),
    out_shape=jax.ShapeDtypeStruct((B, S), jnp.float32),
)


_TB = 512
_UC = 1024


def _ranks_body(nfull_ref, nblk_ref, o_ref):
    i = pl.program_id(0)
    for b in range(B):
        col = nblk_ref[b].reshape(_TB, 1)
        tpos = i * _TB + lax.broadcasted_iota(jnp.int32, (_TB, 1), 0)
        acc = jnp.zeros((_TB,), jnp.int32)
        for cix in range(S // _UC):
            row = nfull_ref[b, pl.ds(cix * _UC, _UC)].reshape(1, _UC)
            upos = cix * _UC + lax.broadcasted_iota(jnp.int32, (_TB, _UC), 1)
            gt = row > col
            eq_lt = (row == col) & (upos < tpos)
            contrib = gt.astype(jnp.int32) + eq_lt.astype(jnp.int32)
            acc = acc + jnp.sum(contrib, axis=1)
        o_ref[b] = acc


_ranks_call = pl.pallas_call(
    _ranks_body,
    grid=(S // _TB,),
    in_specs=[
        pl.BlockSpec((B, S), lambda i: (0, 0)),
        pl.BlockSpec((B, _TB), lambda i: (0, i)),
    ],
    out_specs=pl.BlockSpec((B, _TB), lambda i: (0, i)),
    out_shape=jax.ShapeDtypeStruct((B, S), jnp.int32),
)


_sc_mesh = plsc.VectorSubcoreMesh(core_axis_name="c", subcore_axis_name="s")
_TOK_PER_W = (B * S) // NW


@functools.partial(
    pl.kernel,
    mesh=_sc_mesh,
    out_type=jax.ShapeDtypeStruct((PERM_PAD,), jnp.int32),
    scratch_types=[
        pltpu.VMEM((_TOK_PER_W,), jnp.int32),
        pltpu.VMEM((8, 128), jnp.int32),
        pltpu.VMEM((8, 128), jnp.int32),
        pltpu.SemaphoreType.DMA,
    ],
)
def _perm_kernel(ranks_hbm, perm_hbm, rank_v, dest_v, val_v, sem):
    wid = lax.axis_index("s") * 2 + lax.axis_index("c")
    base = wid * _TOK_PER_W
    pltpu.sync_copy(ranks_hbm.at[pl.ds(base, _TOK_PER_W)], rank_v)
    for i in range(_TOK_PER_W // 16):
        r = rank_v[pl.ds(i * 16, 16)]
        t = base + i * 16 + lax.iota(jnp.int32, 16)
        bb = t >> 13
        slot = bb * K + ((r & 3) << 10) + (r >> 2)
        dump = B * K + (wid << 2)
        dest = jnp.where(r < K, slot, dump)
        dest_v[i // 8, pl.ds((i % 8) * 16, 16)] = dest
        val_v[i // 8, pl.ds((i % 8) * 16, 16)] = t
    copies = [
        pltpu.async_copy(val_v.at[j], perm_hbm.at[dest_v.at[j]], sem)
        for j in range(8)
    ]
    for cp in copies:
        cp.wait()


_ROW_PER_W = (B * K) // NW
_CH = 32


@functools.partial(
    pl.kernel,
    mesh=_sc_mesh,
    out_type=jax.ShapeDtypeStruct((B * K, D), jnp.float32),
    scratch_types=[
        pltpu.VMEM((_CH,), jnp.int32),
        pltpu.VMEM((_CH,), jnp.int32),
        pltpu.VMEM((_CH, D), jnp.float32),
        pltpu.VMEM((_CH, D), jnp.float32),
        pltpu.SemaphoreType.DMA,
        pltpu.SemaphoreType.DMA,
    ],
)
def _gather_kernel(x_hbm, perm_hbm, out_hbm, idx0, idx1, rows0, rows1,
                   sem0, sem1):
    wid = lax.axis_index("s") * 2 + lax.axis_index("c")
    base = wid * _ROW_PER_W
    nch = _ROW_PER_W // _CH
    idx = (idx0, idx1)
    rows = (rows0, rows1)
    sems = (sem0, sem1)
    pltpu.sync_copy(perm_hbm.at[pl.ds(base, _CH)], idx0)
    cp = pltpu.async_copy(x_hbm.at[idx0], rows0, sem0)
    for cix in range(nch):
        cur = cix % 2
        nxt = 1 - cur
        if cix + 1 < nch:
            off = base + (cix + 1) * _CH
            pltpu.sync_copy(perm_hbm.at[pl.ds(off, _CH)], idx[nxt])
            nxt_cp = pltpu.async_copy(x_hbm.at[idx[nxt]], rows[nxt], sems[nxt])
        cp.wait()
        pltpu.sync_copy(rows[cur], out_hbm.at[pl.ds(base + cix * _CH, _CH)])
        if cix + 1 < nch:
            cp = nxt_cp


def _tail_body(g_ref, w1_ref, b1_ref, mk_ref, mem_ref, w2_ref, b2_ref,
               w3_ref, b3_ref, out_ref, upd_ref, h_acc, xc_acc):
    b = pl.program_id(0)
    j = pl.program_id(1)
    part = jnp.dot(g_ref[...], w1_ref[0],
                   preferred_element_type=jnp.float32)

    @pl.when(j == 0)
    def _():
        h_acc[...] = part + b1_ref[...]

    @pl.when(j > 0)
    def _():
        h_acc[...] = h_acc[...] + part

    @pl.when(j == C - 1)
    def _():
        masked = jnp.maximum(h_acc[...] * mk_ref[0], 0.0)
        xc_acc[pl.ds(b, 1), :] = jnp.mean(masked, axis=0).reshape(1, H)

    @pl.when((b == B - 1) & (j == C - 1))
    def _():
        xcomp = xc_acc[...]
        upd = jnp.maximum(
            jnp.dot(xcomp, w2_ref[...], preferred_element_type=jnp.float32)
            + b2_ref[...] + mem_ref[...], 0.0)
        out_ref[...] = jnp.dot(upd, w3_ref[...],
                               preferred_element_type=jnp.float32) + b3_ref[...]
        upd_ref[...] = upd


_tail_call = pl.pallas_call(
    _tail_body,
    grid=(B, C),
    in_specs=[
        pl.BlockSpec((NC, D), lambda b, j: (b * C + j, 0)),
        pl.BlockSpec((1, D, H), lambda b, j: (j, 0, 0)),
        pl.BlockSpec((1, H), lambda b, j: (0, 0)),
        pl.BlockSpec((1, NC, H), lambda b, j: (b, 0, 0)),
        pl.BlockSpec((B, M), lambda b, j: (0, 0)),
        pl.BlockSpec((H, M), lambda b, j: (0, 0)),
        pl.BlockSpec((1, M), lambda b, j: (0, 0)),
        pl.BlockSpec((M, CL), lambda b, j: (0, 0)),
        pl.BlockSpec((1, CL), lambda b, j: (0, 0)),
    ],
    out_specs=[
        pl.BlockSpec((B, CL), lambda b, j: (0, 0)),
        pl.BlockSpec((B, M), lambda b, j: (0, 0)),
    ],
    out_shape=[
        jax.ShapeDtypeStruct((B, CL), jnp.float32),
        jax.ShapeDtypeStruct((B, M), jnp.float32),
    ],
    scratch_shapes=[
        pltpu.VMEM((NC, H), jnp.float32),
        pltpu.VMEM((B, H), jnp.float32),
    ],
)


def kernel(x, memory, W1, b1, W2, b2, W3, b3):
    norms = _norms_call(x)
    ranks = _ranks_call(norms, norms)
    perm = _perm_kernel(ranks.reshape(B * S))
    g = _gather_kernel(x.reshape(B * S, D), perm)
    mask = jax.random.bernoulli(
        jax.random.key(1), 0.5, (B, NC, H)).astype(jnp.float32)
    out, upd = _tail_call(
        g, W1.reshape(C, D, H), b1.reshape(1, H), mask, memory, W2,
        b2.reshape(1, M), W3, b3.reshape(1, CL))
    return (out, upd)


# speedup vs baseline: 3.9706x; 3.9706x over previous
import functools

import jax
import jax.numpy as jnp
from jax import lax
from jax.experimental import pallas as pl
from jax.experimental.pallas import tpu as pltpu
from jax.experimental.pallas import tpu_sc as plsc

B = 4
S = 8192
D = 1024
K = 4096
C = 4
NC = K // C
H = 256
M = 128
CL = 10

NW = 32
PERM_PAD = B * K + B * S


_RB = 512


def _norms_body(x_ref, o_ref):
    y = x_ref[...]
    yy = y * y
    c = yy.reshape(B, _RB, 8, 128)
    acc = c[:, :, 0, :]
    for i in range(1, 8):
        acc = acc + c[:, :, i, :]
    st = 64
    while st >= 1:
        acc = acc[..., :st] + acc[..., st:2 * st]
        st //= 2
    o_ref[...] = jnp.sqrt(acc[..., 0])


_norms_call = pl.pallas_call(
    _norms_body,
    grid=(S // _RB,),
    in_specs=[pl.BlockSpec((B, _RB, D), lambda i: (0, i, 0))],
    out_specs=pl.BlockSpec((B, _RB), lambda i: (0, i)),
    out_shape=jax.ShapeDtypeStruct((B, S), jnp.float32),
)


_TB = 512
_UC = 1024


def _ranks_body(nfull_ref, nblk_ref, o_ref):
    i = pl.program_id(0)
    for b in range(B):
        col = nblk_ref[b].reshape(_TB, 1)
        tpos = i * _TB + lax.broadcasted_iota(jnp.int32, (_TB, 1), 0)
        acc = jnp.zeros((_TB,), jnp.int32)
        for cix in range(S // _UC):
            row = nfull_ref[b, pl.ds(cix * _UC, _UC)].reshape(1, _UC)
            upos = cix * _UC + lax.broadcasted_iota(jnp.int32, (_TB, _UC), 1)
            gt = row > col
            eq_lt = (row == col) & (upos < tpos)
            contrib = gt.astype(jnp.int32) + eq_lt.astype(jnp.int32)
            acc = acc + jnp.sum(contrib, axis=1)
        o_ref[b] = acc


_ranks_call = pl.pallas_call(
    _ranks_body,
    grid=(S // _TB,),
    in_specs=[
        pl.BlockSpec((B, S), lambda i: (0, 0)),
        pl.BlockSpec((B, _TB), lambda i: (0, i)),
    ],
    out_specs=pl.BlockSpec((B, _TB), lambda i: (0, i)),
    out_shape=jax.ShapeDtypeStruct((B, S), jnp.int32),
)


_sc_mesh = plsc.VectorSubcoreMesh(core_axis_name="c", subcore_axis_name="s")
_TOK_PER_W = (B * S) // NW


@functools.partial(
    pl.kernel,
    mesh=_sc_mesh,
    out_type=jax.ShapeDtypeStruct((PERM_PAD,), jnp.int32),
    scratch_types=[
        pltpu.VMEM((_TOK_PER_W,), jnp.int32),
        pltpu.VMEM((8, 128), jnp.int32),
        pltpu.VMEM((8, 128), jnp.int32),
        pltpu.SemaphoreType.DMA,
    ],
)
def _perm_kernel(ranks_hbm, perm_hbm, rank_v, dest_v, val_v, sem):
    wid = lax.axis_index("s") * 2 + lax.axis_index("c")
    base = wid * _TOK_PER_W
    pltpu.sync_copy(ranks_hbm.at[pl.ds(base, _TOK_PER_W)], rank_v)
    for i in range(_TOK_PER_W // 16):
        r = rank_v[pl.ds(i * 16, 16)]
        t = base + i * 16 + lax.iota(jnp.int32, 16)
        bb = t >> 13
        slot = bb * K + ((r & 3) << 10) + (r >> 2)
        dump = B * K + t
        dest = jnp.where(r < K, slot, dump)
        dest_v[i // 8, pl.ds((i % 8) * 16, 16)] = dest
        val_v[i // 8, pl.ds((i % 8) * 16, 16)] = t
    copies = [
        pltpu.async_copy(val_v.at[j], perm_hbm.at[dest_v.at[j]], sem)
        for j in range(8)
    ]
    for cp in copies:
        cp.wait()


_ROW_PER_W = (B * K) // NW
_CH = 32


@functools.partial(
    pl.kernel,
    mesh=_sc_mesh,
    out_type=jax.ShapeDtypeStruct((B * K, D), jnp.float32),
    scratch_types=[
        pltpu.VMEM((_CH,), jnp.int32),
        pltpu.VMEM((_CH,), jnp.int32),
        pltpu.VMEM((_CH, D), jnp.float32),
        pltpu.VMEM((_CH, D), jnp.float32),
        pltpu.SemaphoreType.DMA,
        pltpu.SemaphoreType.DMA,
    ],
)
def _gather_kernel(x_hbm, perm_hbm, out_hbm, idx0, idx1, rows0, rows1,
                   sem0, sem1):
    wid = lax.axis_index("s") * 2 + lax.axis_index("c")
    base = wid * _ROW_PER_W
    nch = _ROW_PER_W // _CH
    idx = (idx0, idx1)
    rows = (rows0, rows1)
    sems = (sem0, sem1)
    pltpu.sync_copy(perm_hbm.at[pl.ds(base, _CH)], idx0)
    cp = pltpu.async_copy(x_hbm.at[idx0], rows0, sem0)
    for cix in range(nch):
        cur = cix % 2
        nxt = 1 - cur
        if cix + 1 < nch:
            off = base + (cix + 1) * _CH
            pltpu.sync_copy(perm_hbm.at[pl.ds(off, _CH)], idx[nxt])
            nxt_cp = pltpu.async_copy(x_hbm.at[idx[nxt]], rows[nxt], sems[nxt])
        cp.wait()
        pltpu.sync_copy(rows[cur], out_hbm.at[pl.ds(base + cix * _CH, _CH)])
        if cix + 1 < nch:
            cp = nxt_cp


def _tail_body(g_ref, w1_ref, b1_ref, mk_ref, mem_ref, w2_ref, b2_ref,
               w3_ref, b3_ref, out_ref, upd_ref, h_acc, xc_acc):
    b = pl.program_id(0)
    j = pl.program_id(1)
    part = jnp.dot(g_ref[...], w1_ref[0],
                   preferred_element_type=jnp.float32)

    @pl.when(j == 0)
    def _():
        h_acc[...] = part + b1_ref[...]

    @pl.when(j > 0)
    def _():
        h_acc[...] = h_acc[...] + part

    @pl.when(j == C - 1)
    def _():
        masked = jnp.maximum(h_acc[...] * mk_ref[0], 0.0)
        xc_acc[pl.ds(b, 1), :] = jnp.mean(masked, axis=0).reshape(1, H)

    @pl.when((b == B - 1) & (j == C - 1))
    def _():
        xcomp = xc_acc[...]
        upd = jnp.maximum(
            jnp.dot(xcomp, w2_ref[...], preferred_element_type=jnp.float32)
            + b2_ref[...] + mem_ref[...], 0.0)
        out_ref[...] = jnp.dot(upd, w3_ref[...],
                               preferred_element_type=jnp.float32) + b3_ref[...]
        upd_ref[...] = upd


_tail_call = pl.pallas_call(
    _tail_body,
    grid=(B, C),
    in_specs=[
        pl.BlockSpec((NC, D), lambda b, j: (b * C + j, 0)),
        pl.BlockSpec((1, D, H), lambda b, j: (j, 0, 0)),
        pl.BlockSpec((1, H), lambda b, j: (0, 0)),
        pl.BlockSpec((1, NC, H), lambda b, j: (b, 0, 0)),
        pl.BlockSpec((B, M), lambda b, j: (0, 0)),
        pl.BlockSpec((H, M), lambda b, j: (0, 0)),
        pl.BlockSpec((1, M), lambda b, j: (0, 0)),
        pl.BlockSpec((M, CL), lambda b, j: (0, 0)),
        pl.BlockSpec((1, CL), lambda b, j: (0, 0)),
    ],
    out_specs=[
        pl.BlockSpec((B, CL), lambda b, j: (0, 0)),
        pl.BlockSpec((B, M), lambda b, j: (0, 0)),
    ],
    out_shape=[
        jax.ShapeDtypeStruct((B, CL), jnp.float32),
        jax.ShapeDtypeStruct((B, M), jnp.float32),
    ],
    scratch_shapes=[
        pltpu.VMEM((NC, H), jnp.float32),
        pltpu.VMEM((B, H), jnp.float32),
    ],
)


def kernel(x, memory, W1, b1, W2, b2, W3, b3):
    norms = _norms_call(x)
    ranks = _ranks_call(norms, norms)
    perm = _perm_kernel(ranks.reshape(B * S))
    g = _gather_kernel(x.reshape(B * S, D), perm)
    mask = jax.random.bernoulli(
        jax.random.key(1), 0.5, (B, NC, H)).astype(jnp.float32)
    out, upd = _tail_call(
        g, W1.reshape(C, D, H), b1.reshape(1, H), mask, memory, W2,
        b2.reshape(1, M), W3, b3.reshape(1, CL))
    return (out, upd)


# speedup vs baseline: 4.5206x; 1.1385x over previous
import functools

import jax
import jax.numpy as jnp
from jax import lax
from jax.experimental import pallas as pl
from jax.experimental.pallas import tpu as pltpu
from jax.experimental.pallas import tpu_sc as plsc

B = 4
S = 8192
D = 1024
K = 4096
C = 4
NC = K // C
H = 256
M = 128
CL = 10

NW = 32
PERM_PAD = B * K + B * S


_RB = 512


def _norms_body(x_ref, o_ref):
    y = x_ref[...]
    yy = y * y
    c = yy.reshape(B, _RB, 8, 128)
    acc = c[:, :, 0, :]
    for i in range(1, 8):
        acc = acc + c[:, :, i, :]
    st = 64
    while st >= 1:
        acc = acc[..., :st] + acc[..., st:2 * st]
        st //= 2
    o_ref[...] = jnp.sqrt(acc[..., 0])


_norms_call = pl.pallas_call(
    _norms_body,
    grid=(S // _RB,),
    in_specs=[pl.BlockSpec((B, _RB, D), lambda i: (0, i, 0))],
    out_specs=pl.BlockSpec((B, _RB), lambda i: (0, i)),
    out_shape=jax.ShapeDtypeStruct((B, S), jnp.float32),
)


_TB = 512
_UC = 512


def _ranks_body(nfull_ref, nblk_ref, o_ref):
    i = pl.program_id(0)
    nt = S // _TB
    for b in range(B):
        col = nblk_ref[b].reshape(_TB, 1)

        def chunk(cix, op):
            row = nfull_ref[b, pl.ds(cix * _UC, _UC)].reshape(1, _UC)
            return jnp.sum(op(row).astype(jnp.float32), axis=1)

        def before(acc, cix):
            return acc + chunk(cix, lambda r: r >= col)

        def after(acc, cix):
            return acc + chunk(cix, lambda r: r > col)

        acc = jnp.zeros((_TB,), jnp.float32)
        acc = lax.fori_loop(0, i, lambda c, a: before(a, c), acc)
        acc = lax.fori_loop(i + 1, nt, lambda c, a: after(a, c), acc)
        row = nfull_ref[b, pl.ds(i * _UC, _UC)].reshape(1, _UC)
        lpos = lax.broadcasted_iota(jnp.int32, (_TB, _UC), 1)
        tloc = lax.broadcasted_iota(jnp.int32, (_TB, 1), 0)
        diag = (row > col) | ((row == col) & (lpos < tloc))
        acc = acc + jnp.sum(diag.astype(jnp.float32), axis=1)
        o_ref[b] = acc.astype(jnp.int32)


_ranks_call = pl.pallas_call(
    _ranks_body,
    grid=(S // _TB,),
    in_specs=[
        pl.BlockSpec((B, S), lambda i: (0, 0)),
        pl.BlockSpec((B, _TB), lambda i: (0, i)),
    ],
    out_specs=pl.BlockSpec((B, _TB), lambda i: (0, i)),
    out_shape=jax.ShapeDtypeStruct((B, S), jnp.int32),
)


_sc_mesh = plsc.VectorSubcoreMesh(core_axis_name="c", subcore_axis_name="s")
_TOK_PER_W = (B * S) // NW


@functools.partial(
    pl.kernel,
    mesh=_sc_mesh,
    out_type=jax.ShapeDtypeStruct((PERM_PAD,), jnp.int32),
    scratch_types=[
        pltpu.VMEM((_TOK_PER_W,), jnp.int32),
        pltpu.VMEM((8, 128), jnp.int32),
        pltpu.VMEM((8, 128), jnp.int32),
        pltpu.SemaphoreType.DMA,
    ],
)
def _perm_kernel(ranks_hbm, perm_hbm, rank_v, dest_v, val_v, sem):
    wid = lax.axis_index("s") * 2 + lax.axis_index("c")
    base = wid * _TOK_PER_W
    pltpu.sync_copy(ranks_hbm.at[pl.ds(base, _TOK_PER_W)], rank_v)
    for i in range(_TOK_PER_W // 16):
        r = rank_v[pl.ds(i * 16, 16)]
        t = base + i * 16 + lax.iota(jnp.int32, 16)
        bb = t >> 13
        slot = bb * K + ((r & 3) << 10) + (r >> 2)
        dump = B * K + t
        dest = jnp.where(r < K, slot, dump)
        dest_v[i // 8, pl.ds((i % 8) * 16, 16)] = dest
        val_v[i // 8, pl.ds((i % 8) * 16, 16)] = t
    copies = [
        pltpu.async_copy(val_v.at[j], perm_hbm.at[dest_v.at[j]], sem)
        for j in range(8)
    ]
    for cp in copies:
        cp.wait()


_ROW_PER_W = (B * K) // NW
_CH = 32


@functools.partial(
    pl.kernel,
    mesh=_sc_mesh,
    out_type=jax.ShapeDtypeStruct((B * K, D), jnp.float32),
    scratch_types=[
        pltpu.VMEM((_CH,), jnp.int32),
        pltpu.VMEM((_CH,), jnp.int32),
        pltpu.VMEM((_CH, D), jnp.float32),
        pltpu.VMEM((_CH, D), jnp.float32),
        pltpu.SemaphoreType.DMA,
        pltpu.SemaphoreType.DMA,
    ],
)
def _gather_kernel(x_hbm, perm_hbm, out_hbm, idx0, idx1, rows0, rows1,
                   sem0, sem1):
    wid = lax.axis_index("s") * 2 + lax.axis_index("c")
    base = wid * _ROW_PER_W
    nch = _ROW_PER_W // _CH
    idx = (idx0, idx1)
    rows = (rows0, rows1)
    sems = (sem0, sem1)
    pltpu.sync_copy(perm_hbm.at[pl.ds(base, _CH)], idx0)
    cp = pltpu.async_copy(x_hbm.at[idx0], rows0, sem0)
    for cix in range(nch):
        cur = cix % 2
        nxt = 1 - cur
        if cix + 1 < nch:
            off = base + (cix + 1) * _CH
            pltpu.sync_copy(perm_hbm.at[pl.ds(off, _CH)], idx[nxt])
            nxt_cp = pltpu.async_copy(x_hbm.at[idx[nxt]], rows[nxt], sems[nxt])
        cp.wait()
        pltpu.sync_copy(rows[cur], out_hbm.at[pl.ds(base + cix * _CH, _CH)])
        if cix + 1 < nch:
            cp = nxt_cp


def _tail_body(g_ref, w1_ref, b1_ref, mk_ref, mem_ref, w2_ref, b2_ref,
               w3_ref, b3_ref, out_ref, upd_ref, h_acc, xc_acc):
    b = pl.program_id(0)
    j = pl.program_id(1)
    part = jnp.dot(g_ref[...], w1_ref[0], precision=lax.Precision.DEFAULT,
                   preferred_element_type=jnp.float32)

    @pl.when(j == 0)
    def _():
        h_acc[...] = part + b1_ref[...]

    @pl.when(j > 0)
    def _():
        h_acc[...] = h_acc[...] + part

    @pl.when(j == C - 1)
    def _():
        masked = jnp.maximum(h_acc[...] * mk_ref[0], 0.0)
        xc_acc[pl.ds(b, 1), :] = jnp.mean(masked, axis=0).reshape(1, H)

    @pl.when((b == B - 1) & (j == C - 1))
    def _():
        xcomp = xc_acc[...]
        upd = jnp.maximum(
            jnp.dot(xcomp, w2_ref[...], precision=lax.Precision.DEFAULT,
                    preferred_element_type=jnp.float32)
            + b2_ref[...] + mem_ref[...], 0.0)
        out_ref[...] = jnp.dot(upd, w3_ref[...], precision=lax.Precision.DEFAULT,
                               preferred_element_type=jnp.float32) + b3_ref[...]
        upd_ref[...] = upd


_tail_call = pl.pallas_call(
    _tail_body,
    grid=(B, C),
    in_specs=[
        pl.BlockSpec((NC, D), lambda b, j: (b * C + j, 0)),
        pl.BlockSpec((1, D, H), lambda b, j: (j, 0, 0)),
        pl.BlockSpec((1, H), lambda b, j: (0, 0)),
        pl.BlockSpec((1, NC, H), lambda b, j: (b, 0, 0)),
        pl.BlockSpec((B, M), lambda b, j: (0, 0)),
        pl.BlockSpec((H, M), lambda b, j: (0, 0)),
        pl.BlockSpec((1, M), lambda b, j: (0, 0)),
        pl.BlockSpec((M, CL), lambda b, j: (0, 0)),
        pl.BlockSpec((1, CL), lambda b, j: (0, 0)),
    ],
    out_specs=[
        pl.BlockSpec((B, CL), lambda b, j: (0, 0)),
        pl.BlockSpec((B, M), lambda b, j: (0, 0)),
    ],
    out_shape=[
        jax.ShapeDtypeStruct((B, CL), jnp.float32),
        jax.ShapeDtypeStruct((B, M), jnp.float32),
    ],
    scratch_shapes=[
        pltpu.VMEM((NC, H), jnp.float32),
        pltpu.VMEM((B, H), jnp.float32),
    ],
)


def kernel(x, memory, W1, b1, W2, b2, W3, b3):
    norms = _norms_call(x)
    ranks = _ranks_call(norms, norms)
    perm = _perm_kernel(ranks.reshape(B * S))
    g = _gather_kernel(x.reshape(B * S, D), perm)
    mask = jax.random.bernoulli(
        jax.random.key(1), 0.5, (B, NC, H)).astype(jnp.float32)
    out, upd = _tail_call(
        g, W1.reshape(C, D, H), b1.reshape(1, H), mask, memory, W2,
        b2.reshape(1, M), W3, b3.reshape(1, CL))
    return (out, upd)
